# baseline (speedup 1.0000x reference)
import functools

import jax
import jax.numpy as jnp
from jax import lax
from jax.experimental import pallas as pl
from jax.experimental.pallas import tpu as pltpu
from jax.experimental.pallas import tpu_sc as plsc

_BLK = 512
_NSTREAM = 2
_NC, _NS, _L = 2, 16, 16
_NW = _NC * _NS
_K = 10


def _logits_block(*refs):
    x_refs = refs[:_NSTREAM]
    wt_ref, b_ref = refs[_NSTREAM], refs[_NSTREAM + 1]
    o_refs = refs[_NSTREAM + 2:]
    for x_ref, o_ref in zip(x_refs, o_refs):
        logits = jnp.dot(x_ref[...], wt_ref[...],
                         preferred_element_type=jnp.float32) + b_ref[...]
        o_ref[...] = logits.T


def _route_sc(lgt_hbm, o0_hbm, o1_hbm, lbuf, o0buf, o1buf):
    wid = lax.axis_index("s") * _NC + lax.axis_index("c")
    m_total = lgt_hbm.shape[1]
    tpw = m_total // _NW
    base = wid * tpw
    pltpu.sync_copy(lgt_hbm.at[:, pl.ds(base, tpw)], lbuf)
    for g in range(tpw // _L):
        off = g * _L
        lv0 = lbuf[0, pl.ds(off, _L)]
        m = jnp.maximum(lbuf[1, pl.ds(off, _L)], lbuf[2, pl.ds(off, _L)])
        for k in range(3, _K):
            m = jnp.maximum(m, lbuf[k, pl.ds(off, _L)])
        f0 = jnp.where(lv0 >= m, jnp.float32(1.0), jnp.float32(0.0))
        o0buf[pl.ds(off, _L)] = f0
        o1buf[pl.ds(off, _L)] = 1.0 - f0
    pltpu.sync_copy(o0buf, o0_hbm.at[pl.ds(base, tpw)])
    pltpu.sync_copy(o1buf, o1_hbm.at[pl.ds(base, tpw)])


@jax.jit
def kernel(x, W, b):
    B, S, D = x.shape
    K = W.shape[0]
    M = B * S
    H = M // _NSTREAM
    nb = H // _BLK
    x2 = x.reshape(M, D)
    wt = W.T
    b2 = b.reshape(1, K)

    def make_in(s):
        return pl.BlockSpec((_BLK, D), lambda i, s=s: (i + s * nb, 0))

    lgts = pl.pallas_call(
        _logits_block,
        grid=(nb,),
        in_specs=[make_in(s) for s in range(_NSTREAM)] + [
            pl.BlockSpec((D, K), lambda i: (0, 0)),
            pl.BlockSpec((1, K), lambda i: (0, 0)),
        ],
        out_specs=[
            pl.BlockSpec((K, _BLK), lambda i: (0, i))
            for _ in range(_NSTREAM)
        ],
        out_shape=[
            jax.ShapeDtypeStruct((K, H), jnp.float32)
            for _ in range(_NSTREAM)
        ],
        compiler_params=pltpu.CompilerParams(
            dimension_semantics=("arbitrary",),
        ),
    )(*([x2] * _NSTREAM), wt, b2)
    lgt = jnp.concatenate(lgts, axis=1)

    tpw = M // _NW
    o0, o1 = pl.kernel(
        _route_sc,
        out_type=[
            jax.ShapeDtypeStruct((M,), jnp.float32),
            jax.ShapeDtypeStruct((M,), jnp.float32),
        ],
        scratch_types=[
            pltpu.VMEM((K, tpw), jnp.float32),
            pltpu.VMEM((tpw,), jnp.float32),
            pltpu.VMEM((tpw,), jnp.float32),
        ],
        mesh=plsc.VectorSubcoreMesh(core_axis_name="c", subcore_axis_name="s"),
    )(lgt)
    return jnp.stack([o0, o1], axis=-1).reshape(B, S, 2)


# speedup vs baseline: 1.1527x; 1.1527x over previous
import jax
import jax.numpy as jnp
from jax.experimental import pallas as pl
from jax.experimental.pallas import tpu as pltpu

_BLK = 512
_NSTREAM = 2


def _gate_block(*refs):
    x_refs = refs[:_NSTREAM]
    wt_ref, b_ref = refs[_NSTREAM], refs[_NSTREAM + 1]
    o_refs = refs[_NSTREAM + 2:]
    for x_ref, o_ref in zip(x_refs, o_refs):
        logits = jnp.dot(x_ref[...], wt_ref[...],
                         preferred_element_type=jnp.float32) + b_ref[...]
        l0 = logits[:, 0:1]
        lrest = jnp.max(logits[:, 1:], axis=1, keepdims=True)
        is0 = (l0 >= lrest).astype(jnp.float32)
        o_ref[...] = jnp.concatenate([is0, 1.0 - is0], axis=1)


@jax.jit
def kernel(x, W, b):
    B, S, D = x.shape
    K = W.shape[0]
    M = B * S
    H = M // _NSTREAM
    nb = H // _BLK
    x2 = x.reshape(M, D)
    wt = W.T
    b2 = b.reshape(1, K)

    def make_in(s):
        return pl.BlockSpec((_BLK, D), lambda i, s=s: (i + s * nb, 0))

    outs = pl.pallas_call(
        _gate_block,
        grid=(nb,),
        in_specs=[make_in(s) for s in range(_NSTREAM)] + [
            pl.BlockSpec((D, K), lambda i: (0, 0)),
            pl.BlockSpec((1, K), lambda i: (0, 0)),
        ],
        out_specs=[
            pl.BlockSpec((_BLK, 2), lambda i: (i, 0))
            for _ in range(_NSTREAM)
        ],
        out_shape=[
            jax.ShapeDtypeStruct((H, 2), jnp.float32)
            for _ in range(_NSTREAM)
        ],
        compiler_params=pltpu.CompilerParams(
            dimension_semantics=("arbitrary",),
        ),
    )(*([x2] * _NSTREAM), wt, b2)
    return jnp.concatenate(outs, axis=0).reshape(B, S, 2)
